# Initial kernel scaffold; baseline (speedup 1.0000x reference)
import jax
import jax.numpy as jnp
from jax.experimental import pallas as pl


def kernel(x, obj_dict, bg_dict, rand_inds, conv1_w, bn1_g, bn1_b, to_q_w, to_kv_w, to_out_w, to_out_b, conv2_w, bn2_g, bn2_b):
    raise NotImplementedError("write your pallas kernel here")


# speedup vs baseline: 1.5089x; 1.5089x over previous
import functools

import jax
import jax.numpy as jnp
from jax import lax
from jax.experimental import pallas as pl
from jax.experimental.pallas import tpu as pltpu
from jax.experimental.pallas import tpu_sc as plsc

CH = 128
EMB = 64
HEADS = 2
CROP = 300
CH_H = EMB // HEADS
NPIX = CROP * CROP
NUM_OBJ = CROP // 2
HALF = NPIX // 2

P_BLK = 3600
N_PBLK = NPIX // P_BLK
P_BLK2 = 4096
N_PBLK2 = -(-NPIX // P_BLK2)
G_BLK = 8
NGROUP = HEADS * CROP
CW = 128

NW = 32
ROWS = HEADS * NPIX
ROWS_PER_W = 5632
ROWS_PAD = NW * ROWS_PER_W
SC_CHUNK = 512


def _conv1_body(x_ref, w_ref, y_ref, stats_ref):
    j = pl.program_id(0)
    y = lax.dot_general(x_ref[...], w_ref[...], (((0,), (1,)), ((), ())),
                        preferred_element_type=jnp.float32)
    y_ref[...] = y
    row = lax.broadcasted_iota(jnp.int32, (P_BLK2, 1), 0) + j * P_BLK2
    ym = jnp.where(row < NPIX, y, 0.0)
    st = jnp.stack([jnp.sum(ym, axis=0), jnp.sum(ym * ym, axis=0)], axis=0)

    @pl.when(j == 0)
    def _():
        stats_ref[...] = st

    @pl.when(j > 0)
    def _():
        stats_ref[...] = stats_ref[...] + st


def _bn1_body(y_ref, stats_ref, g_ref, b_ref, out_ref):
    st = stats_ref[...]
    m = st[0:1, :] * (1.0 / NPIX)
    v = st[1:2, :] * (1.0 / NPIX) - m * m
    scale = g_ref[...] * lax.rsqrt(v + 1e-5)
    shift = b_ref[...] - m * scale
    yn = jnp.maximum(y_ref[...] * scale + shift, 0.0)
    z = jnp.zeros((yn.shape[0], CW - CH_H), jnp.float32)
    out_ref[0] = jnp.concatenate([yn[:, :CH_H], z], axis=1)
    out_ref[1] = jnp.concatenate([yn[:, CH_H:], z], axis=1)


def _attn_body(s_ref, wq_ref, wkv_ref, o_ref):
    s = s_ref[...][:, :CH_H]
    q = lax.dot_general(s, wq_ref[...], (((1,), (1,)), ((), ())),
                        preferred_element_type=jnp.float32)
    kv = lax.dot_general(s, wkv_ref[...], (((1,), (1,)), ((), ())),
                         preferred_element_type=jnp.float32)
    q3 = q.reshape(G_BLK, CROP, CH_H)
    k3 = kv[:, :CH_H].reshape(G_BLK, CROP, CH_H)
    v3 = kv[:, CH_H:].reshape(G_BLK, CROP, CH_H)
    dots = lax.dot_general(q3, k3, (((2,), (2,)), ((0,), (0,))),
                           preferred_element_type=jnp.float32)
    dots = dots * (CH_H ** -0.5)
    mx = jnp.max(dots, axis=2, keepdims=True)
    e = jnp.exp(dots - mx)
    attn = e / jnp.sum(e, axis=2, keepdims=True)
    o_ref[...] = lax.dot_general(attn, v3, (((2,), (1,)), ((0,), (0,))),
                                 preferred_element_type=jnp.float32)


def _outconv2_body(a0_ref, a1_ref, x10_ref, x11_ref, wout_ref, bout_ref,
                   w2_ref, z_ref, stats_ref):
    j = pl.program_id(0)
    img = jnp.concatenate([a0_ref[...][:, :CH_H], a1_ref[...][:, :CH_H]],
                          axis=1)
    y = lax.dot_general(img, wout_ref[...], (((1,), (1,)), ((), ())),
                        preferred_element_type=jnp.float32) + bout_ref[...]
    y = jnp.maximum(y, 0.0)
    x1 = jnp.concatenate([x10_ref[...][:, :CH_H], x11_ref[...][:, :CH_H]],
                         axis=1)
    w2 = w2_ref[...]
    z = (lax.dot_general(y, w2[:, :EMB], (((1,), (1,)), ((), ())),
                         preferred_element_type=jnp.float32)
         + lax.dot_general(x1, w2[:, EMB:], (((1,), (1,)), ((), ())),
                           preferred_element_type=jnp.float32))
    z_ref[...] = z
    st = jnp.stack([jnp.sum(z, axis=0), jnp.sum(z * z, axis=0)], axis=0)

    @pl.when(j == 0)
    def _():
        stats_ref[...] = st

    @pl.when(j > 0)
    def _():
        stats_ref[...] = stats_ref[...] + st


def _bn2_body(z_ref, stats_ref, g_ref, b_ref, out_ref):
    st = stats_ref[...]
    m = st[0:1, :] * (1.0 / NPIX)
    v = st[1:2, :] * (1.0 / NPIX) - m * m
    scale = g_ref[...] * lax.rsqrt(v + 1e-5)
    shift = b_ref[...] - m * scale
    zn = jnp.maximum(z_ref[...] * scale + shift, 0.0)
    out_ref[...] = zn.T


def _sc_worker_id():
    return lax.axis_index("s") * 2 + lax.axis_index("c")


@functools.lru_cache(maxsize=None)
def _sc_kernels():
    mesh = plsc.VectorSubcoreMesh(core_axis_name="c", subcore_axis_name="s")
    common = dict(
        mesh=mesh,
        out_type=jax.ShapeDtypeStruct((ROWS_PAD, CW), jnp.float32),
        scratch_types=[
            pltpu.VMEM((SC_CHUNK,), jnp.int32),
            pltpu.VMEM((SC_CHUNK, CW), jnp.float32),
            pltpu.SemaphoreType.DMA,
        ],
    )

    @functools.partial(pl.kernel, **common)
    def sc_gather(table_hbm, idx_hbm, out_hbm, idx_v, rows_v, sem):
        base = _sc_worker_id() * ROWS_PER_W
        for i in range(ROWS_PER_W // SC_CHUNK):
            off = base + i * SC_CHUNK
            pltpu.sync_copy(idx_hbm.at[pl.ds(off, SC_CHUNK)], idx_v)
            pltpu.async_copy(table_hbm.at[idx_v], rows_v, sem).wait()
            pltpu.sync_copy(rows_v, out_hbm.at[pl.ds(off, SC_CHUNK)])

    @functools.partial(pl.kernel, **common)
    def sc_scatter(vals_hbm, idx_hbm, out_hbm, idx_v, rows_v, sem):
        base = _sc_worker_id() * ROWS_PER_W
        for i in range(ROWS_PER_W // SC_CHUNK):
            off = base + i * SC_CHUNK
            pltpu.sync_copy(idx_hbm.at[pl.ds(off, SC_CHUNK)], idx_v)
            pltpu.sync_copy(vals_hbm.at[pl.ds(off, SC_CHUNK)], rows_v)
            pltpu.async_copy(rows_v, out_hbm.at[idx_v], sem).wait()

    return sc_gather, sc_scatter


def _pixel_indices(obj_dict, bg_dict, rand_inds):
    dict_cat = jnp.concatenate([obj_dict, bg_dict])
    row_off = (jnp.arange(CROP) >= NUM_OBJ).astype(jnp.int32) * HALF
    adj = rand_inds + row_off[None, :, None]
    pix = jnp.take(dict_cat, adj.reshape(-1)).reshape(HEADS, CROP, CROP)
    head_off = (jnp.arange(HEADS, dtype=jnp.int32) * NPIX)[:, None, None]
    return (pix + head_off).reshape(-1)


def kernel(x, obj_dict, bg_dict, rand_inds, conv1_w, bn1_g, bn1_b,
           to_q_w, to_kv_w, to_out_w, to_out_b, conv2_w, bn2_g, bn2_b):
    x_fm = x.reshape(CH, NPIX)

    y, stats1 = pl.pallas_call(
        _conv1_body,
        grid=(N_PBLK2,),
        in_specs=[
            pl.BlockSpec((CH, P_BLK2), lambda j: (0, j)),
            pl.BlockSpec((EMB, CH), lambda j: (0, 0)),
        ],
        out_specs=[
            pl.BlockSpec((P_BLK2, EMB), lambda j: (j, 0)),
            pl.BlockSpec((2, EMB), lambda j: (0, 0)),
        ],
        out_shape=[
            jax.ShapeDtypeStruct((NPIX, EMB), jnp.float32),
            jax.ShapeDtypeStruct((2, EMB), jnp.float32),
        ],
    )(x_fm, conv1_w)

    x1_slab = pl.pallas_call(
        _bn1_body,
        grid=(N_PBLK,),
        in_specs=[
            pl.BlockSpec((P_BLK, EMB), lambda j: (j, 0)),
            pl.BlockSpec((2, EMB), lambda j: (0, 0)),
            pl.BlockSpec((1, EMB), lambda j: (0, 0)),
            pl.BlockSpec((1, EMB), lambda j: (0, 0)),
        ],
        out_specs=pl.BlockSpec((HEADS, P_BLK, CW), lambda j: (0, j, 0)),
        out_shape=jax.ShapeDtypeStruct((HEADS, NPIX, CW), jnp.float32),
    )(y, stats1, bn1_g.reshape(1, EMB), bn1_b.reshape(1, EMB))

    idx_all = _pixel_indices(obj_dict, bg_dict, rand_inds)
    n_pad = ROWS_PAD - ROWS
    idx_g = jnp.concatenate([idx_all, jnp.zeros((n_pad,), jnp.int32)])
    idx_s = jnp.concatenate(
        [idx_all, ROWS + jnp.arange(n_pad, dtype=jnp.int32)])

    sc_gather, sc_scatter = _sc_kernels()
    table = x1_slab.reshape(ROWS, CW)
    seq = sc_gather(table, idx_g)

    o = pl.pallas_call(
        _attn_body,
        grid=(NGROUP // G_BLK,),
        in_specs=[
            pl.BlockSpec((G_BLK * CROP, CW), lambda j: (j, 0)),
            pl.BlockSpec((CH_H, CH_H), lambda j: (0, 0)),
            pl.BlockSpec((2 * CH_H, CH_H), lambda j: (0, 0)),
        ],
        out_specs=pl.BlockSpec((G_BLK, CROP, CH_H), lambda j: (j, 0, 0)),
        out_shape=jax.ShapeDtypeStruct((NGROUP, CROP, CH_H), jnp.float32),
    )(seq, to_q_w, to_kv_w)

    vals = o.reshape(NGROUP, CH_H, CROP).transpose(0, 2, 1).reshape(ROWS, CH_H)
    vals = jnp.pad(vals, ((0, n_pad), (0, CW - CH_H)))

    table2 = sc_scatter(vals, idx_s)

    z, stats2 = pl.pallas_call(
        _outconv2_body,
        grid=(N_PBLK,),
        in_specs=[
            pl.BlockSpec((P_BLK, CW), lambda j: (j, 0)),
            pl.BlockSpec((P_BLK, CW), lambda j: (j + N_PBLK, 0)),
            pl.BlockSpec((P_BLK, CW), lambda j: (j, 0)),
            pl.BlockSpec((P_BLK, CW), lambda j: (j + N_PBLK, 0)),
            pl.BlockSpec((EMB, EMB), lambda j: (0, 0)),
            pl.BlockSpec((1, EMB), lambda j: (0, 0)),
            pl.BlockSpec((EMB, 2 * EMB), lambda j: (0, 0)),
        ],
        out_specs=[
            pl.BlockSpec((P_BLK, EMB), lambda j: (j, 0)),
            pl.BlockSpec((2, EMB), lambda j: (0, 0)),
        ],
        out_shape=[
            jax.ShapeDtypeStruct((NPIX, EMB), jnp.float32),
            jax.ShapeDtypeStruct((2, EMB), jnp.float32),
        ],
    )(table2, table2, table, table, to_out_w, to_out_b.reshape(1, EMB),
      conv2_w)

    out = pl.pallas_call(
        _bn2_body,
        grid=(N_PBLK2,),
        in_specs=[
            pl.BlockSpec((P_BLK2, EMB), lambda j: (j, 0)),
            pl.BlockSpec((2, EMB), lambda j: (0, 0)),
            pl.BlockSpec((1, EMB), lambda j: (0, 0)),
            pl.BlockSpec((1, EMB), lambda j: (0, 0)),
        ],
        out_specs=pl.BlockSpec((EMB, P_BLK2), lambda j: (0, j)),
        out_shape=jax.ShapeDtypeStruct((EMB, NPIX), jnp.float32),
    )(z, stats2, bn2_g.reshape(1, EMB), bn2_b.reshape(1, EMB))

    return out.reshape(1, EMB, CROP, CROP)


# speedup vs baseline: 1.5348x; 1.0171x over previous
import functools

import jax
import jax.numpy as jnp
from jax import lax
from jax.experimental import pallas as pl
from jax.experimental.pallas import tpu as pltpu
from jax.experimental.pallas import tpu_sc as plsc

CH = 128
EMB = 64
HEADS = 2
CROP = 300
CH_H = EMB // HEADS
NPIX = CROP * CROP
NUM_OBJ = CROP // 2
HALF = NPIX // 2

P_BLK = 3600
N_PBLK = NPIX // P_BLK
P_BLK2 = 4096
N_PBLK2 = -(-NPIX // P_BLK2)
G_BLK = 8
NGROUP = HEADS * CROP
CW = 128

NW = 32
ROWS = HEADS * NPIX
ROWS_PER_W = 5632
ROWS_PAD = NW * ROWS_PER_W
SC_CHUNK = 512


def _conv1_body(x_ref, w_ref, slab_ref, stats_ref):
    j = pl.program_id(0)
    y = lax.dot_general(x_ref[...], w_ref[...], (((0,), (1,)), ((), ())),
                        preferred_element_type=jnp.float32)
    z = jnp.zeros((y.shape[0], CW - CH_H), jnp.float32)
    slab_ref[0] = jnp.concatenate([y[:, :CH_H], z], axis=1)
    slab_ref[1] = jnp.concatenate([y[:, CH_H:], z], axis=1)
    row = lax.broadcasted_iota(jnp.int32, (P_BLK2, 1), 0) + j * P_BLK2
    ym = jnp.where(row < NPIX, y, 0.0)
    st = jnp.stack([jnp.sum(ym, axis=0), jnp.sum(ym * ym, axis=0)], axis=0)

    @pl.when(j == 0)
    def _():
        stats_ref[...] = st

    @pl.when(j > 0)
    def _():
        stats_ref[...] = stats_ref[...] + st


def _attn_body(s_ref, sc_ref, sh_ref, wq_ref, wkv_ref, v_ref):
    raw = s_ref[...][:, :CH_H].reshape(G_BLK, CROP, CH_H)
    s3 = jnp.maximum(raw * sc_ref[...][:, None, :] + sh_ref[...][:, None, :],
                     0.0)
    s = s3.reshape(G_BLK * CROP, CH_H)
    q = lax.dot_general(s, wq_ref[...], (((1,), (1,)), ((), ())),
                        preferred_element_type=jnp.float32)
    kv = lax.dot_general(s, wkv_ref[...], (((1,), (1,)), ((), ())),
                         preferred_element_type=jnp.float32)
    q3 = q.reshape(G_BLK, CROP, CH_H)
    k3 = kv[:, :CH_H].reshape(G_BLK, CROP, CH_H)
    v3 = kv[:, CH_H:].reshape(G_BLK, CROP, CH_H)
    dots = lax.dot_general(q3, k3, (((2,), (2,)), ((0,), (0,))),
                           preferred_element_type=jnp.float32)
    dots = dots * (CH_H ** -0.5)
    mx = jnp.max(dots, axis=2, keepdims=True)
    e = jnp.exp(dots - mx)
    attn = e / jnp.sum(e, axis=2, keepdims=True)
    v_ref[...] = lax.dot_general(attn, v3, (((2,), (1,)), ((0,), (0,))),
                                 preferred_element_type=jnp.float32)


def _outconv2_body(a0_ref, a1_ref, y0_ref, y1_ref, sc_ref, sh_ref,
                   wout_ref, bout_ref, w2_ref, z_ref, stats_ref):
    j = pl.program_id(0)
    img = jnp.concatenate([a0_ref[...][:, :CH_H], a1_ref[...][:, :CH_H]],
                          axis=1)
    y = lax.dot_general(img, wout_ref[...], (((1,), (1,)), ((), ())),
                        preferred_element_type=jnp.float32) + bout_ref[...]
    y = jnp.maximum(y, 0.0)
    yraw = jnp.concatenate([y0_ref[...][:, :CH_H], y1_ref[...][:, :CH_H]],
                           axis=1)
    x1 = jnp.maximum(yraw * sc_ref[...] + sh_ref[...], 0.0)
    w2 = w2_ref[...]
    z = (lax.dot_general(y, w2[:, :EMB], (((1,), (1,)), ((), ())),
                         preferred_element_type=jnp.float32)
         + lax.dot_general(x1, w2[:, EMB:], (((1,), (1,)), ((), ())),
                           preferred_element_type=jnp.float32))
    z_ref[...] = z
    st = jnp.stack([jnp.sum(z, axis=0), jnp.sum(z * z, axis=0)], axis=0)

    @pl.when(j == 0)
    def _():
        stats_ref[...] = st

    @pl.when(j > 0)
    def _():
        stats_ref[...] = stats_ref[...] + st


def _bn2_body(z_ref, stats_ref, g_ref, b_ref, out_ref):
    st = stats_ref[...]
    m = st[0:1, :] * (1.0 / NPIX)
    v = st[1:2, :] * (1.0 / NPIX) - m * m
    scale = g_ref[...] * lax.rsqrt(v + 1e-5)
    shift = b_ref[...] - m * scale
    zn = jnp.maximum(z_ref[...] * scale + shift, 0.0)
    out_ref[...] = zn.T


def _sc_worker_id():
    return lax.axis_index("s") * 2 + lax.axis_index("c")


@functools.lru_cache(maxsize=None)
def _sc_kernels():
    mesh = plsc.VectorSubcoreMesh(core_axis_name="c", subcore_axis_name="s")
    common = dict(
        mesh=mesh,
        out_type=jax.ShapeDtypeStruct((ROWS_PAD, CW), jnp.float32),
        scratch_types=[
            pltpu.VMEM((SC_CHUNK,), jnp.int32),
            pltpu.VMEM((SC_CHUNK, CW), jnp.float32),
            pltpu.SemaphoreType.DMA,
        ],
    )

    @functools.partial(pl.kernel, **common)
    def sc_gather(table_hbm, idx_hbm, out_hbm, idx_v, rows_v, sem):
        base = _sc_worker_id() * ROWS_PER_W
        for i in range(ROWS_PER_W // SC_CHUNK):
            off = base + i * SC_CHUNK
            pltpu.sync_copy(idx_hbm.at[pl.ds(off, SC_CHUNK)], idx_v)
            pltpu.async_copy(table_hbm.at[idx_v], rows_v, sem).wait()
            pltpu.sync_copy(rows_v, out_hbm.at[pl.ds(off, SC_CHUNK)])

    @functools.partial(pl.kernel, **common)
    def sc_scatter(vals_hbm, idx_hbm, out_hbm, idx_v, rows_v, sem):
        base = _sc_worker_id() * ROWS_PER_W
        for i in range(ROWS_PER_W // SC_CHUNK):
            off = base + i * SC_CHUNK
            pltpu.sync_copy(idx_hbm.at[pl.ds(off, SC_CHUNK)], idx_v)
            pltpu.sync_copy(vals_hbm.at[pl.ds(off, SC_CHUNK)], rows_v)
            pltpu.async_copy(rows_v, out_hbm.at[idx_v], sem).wait()

    return sc_gather, sc_scatter


def _pixel_indices(obj_dict, bg_dict, rand_inds):
    dict_cat = jnp.concatenate([obj_dict, bg_dict])
    row_off = (jnp.arange(CROP) >= NUM_OBJ).astype(jnp.int32) * HALF
    adj = rand_inds + row_off[None, :, None]
    pix = jnp.take(dict_cat, adj.reshape(-1)).reshape(HEADS, CROP, CROP)
    head_off = (jnp.arange(HEADS, dtype=jnp.int32) * NPIX)[:, None, None]
    return (pix + head_off).reshape(-1)


def kernel(x, obj_dict, bg_dict, rand_inds, conv1_w, bn1_g, bn1_b,
           to_q_w, to_kv_w, to_out_w, to_out_b, conv2_w, bn2_g, bn2_b):
    x_fm = x.reshape(CH, NPIX)

    slab, stats1 = pl.pallas_call(
        _conv1_body,
        grid=(N_PBLK2,),
        in_specs=[
            pl.BlockSpec((CH, P_BLK2), lambda j: (0, j)),
            pl.BlockSpec((EMB, CH), lambda j: (0, 0)),
        ],
        out_specs=[
            pl.BlockSpec((HEADS, P_BLK2, CW), lambda j: (0, j, 0)),
            pl.BlockSpec((2, EMB), lambda j: (0, 0)),
        ],
        out_shape=[
            jax.ShapeDtypeStruct((HEADS, NPIX, CW), jnp.float32),
            jax.ShapeDtypeStruct((2, EMB), jnp.float32),
        ],
    )(x_fm, conv1_w)

    m1 = stats1[0] * (1.0 / NPIX)
    v1 = stats1[1] * (1.0 / NPIX) - m1 * m1
    sc1 = bn1_g * lax.rsqrt(v1 + 1e-5)
    sh1 = bn1_b - m1 * sc1
    head_of_g = (jnp.arange(NGROUP) // CROP).astype(jnp.int32)
    sc_g = sc1.reshape(HEADS, CH_H)[head_of_g]
    sh_g = sh1.reshape(HEADS, CH_H)[head_of_g]

    idx_all = _pixel_indices(obj_dict, bg_dict, rand_inds)
    n_pad = ROWS_PAD - ROWS
    idx_g = jnp.concatenate([idx_all, jnp.zeros((n_pad,), jnp.int32)])
    idx_s = jnp.concatenate(
        [idx_all, ROWS + jnp.arange(n_pad, dtype=jnp.int32)])

    sc_gather, sc_scatter = _sc_kernels()
    table = slab.reshape(ROWS, CW)
    seq = sc_gather(table, idx_g)

    o = pl.pallas_call(
        _attn_body,
        grid=(NGROUP // G_BLK,),
        in_specs=[
            pl.BlockSpec((G_BLK * CROP, CW), lambda j: (j, 0)),
            pl.BlockSpec((G_BLK, CH_H), lambda j: (j, 0)),
            pl.BlockSpec((G_BLK, CH_H), lambda j: (j, 0)),
            pl.BlockSpec((CH_H, CH_H), lambda j: (0, 0)),
            pl.BlockSpec((2 * CH_H, CH_H), lambda j: (0, 0)),
        ],
        out_specs=pl.BlockSpec((G_BLK, CROP, CH_H), lambda j: (j, 0, 0)),
        out_shape=jax.ShapeDtypeStruct((NGROUP, CROP, CH_H), jnp.float32),
    )(seq, sc_g, sh_g, to_q_w, to_kv_w)

    vals = o.reshape(NGROUP, CH_H, CROP).transpose(0, 2, 1).reshape(ROWS, CH_H)
    vals = jnp.pad(vals, ((0, n_pad), (0, CW - CH_H)))

    table2 = sc_scatter(vals, idx_s)

    z, stats2 = pl.pallas_call(
        _outconv2_body,
        grid=(N_PBLK,),
        in_specs=[
            pl.BlockSpec((P_BLK, CW), lambda j: (j, 0)),
            pl.BlockSpec((P_BLK, CW), lambda j: (j + N_PBLK, 0)),
            pl.BlockSpec((P_BLK, CW), lambda j: (j, 0)),
            pl.BlockSpec((P_BLK, CW), lambda j: (j + N_PBLK, 0)),
            pl.BlockSpec((1, EMB), lambda j: (0, 0)),
            pl.BlockSpec((1, EMB), lambda j: (0, 0)),
            pl.BlockSpec((EMB, EMB), lambda j: (0, 0)),
            pl.BlockSpec((1, EMB), lambda j: (0, 0)),
            pl.BlockSpec((EMB, 2 * EMB), lambda j: (0, 0)),
        ],
        out_specs=[
            pl.BlockSpec((P_BLK, EMB), lambda j: (j, 0)),
            pl.BlockSpec((2, EMB), lambda j: (0, 0)),
        ],
        out_shape=[
            jax.ShapeDtypeStruct((NPIX, EMB), jnp.float32),
            jax.ShapeDtypeStruct((2, EMB), jnp.float32),
        ],
    )(table2, table2, table, table, sc1.reshape(1, EMB), sh1.reshape(1, EMB),
      to_out_w, to_out_b.reshape(1, EMB), conv2_w)

    out = pl.pallas_call(
        _bn2_body,
        grid=(N_PBLK2,),
        in_specs=[
            pl.BlockSpec((P_BLK2, EMB), lambda j: (j, 0)),
            pl.BlockSpec((2, EMB), lambda j: (0, 0)),
            pl.BlockSpec((1, EMB), lambda j: (0, 0)),
            pl.BlockSpec((1, EMB), lambda j: (0, 0)),
        ],
        out_specs=pl.BlockSpec((EMB, P_BLK2), lambda j: (0, j)),
        out_shape=jax.ShapeDtypeStruct((EMB, NPIX), jnp.float32),
    )(z, stats2, bn2_g.reshape(1, EMB), bn2_b.reshape(1, EMB))

    return out.reshape(1, EMB, CROP, CROP)
